# baseline (speedup 1.0000x reference)
import functools

import jax
import jax.numpy as jnp
from jax import lax
from jax.experimental import pallas as pl
from jax.experimental.pallas import tpu as pltpu
from jax.experimental.pallas import tpu_sc as plsc

B = 16384
K = 32
FEAT = 128
EMB = 128
NC, NS = 2, 16
NW = NC * NS
SEEDS_PER_W = B // NW
CHUNK = 128
NCH = SEEDS_PER_W // CHUNK


def _sc_gather_sum(idx, features, nb):
    mesh = plsc.VectorSubcoreMesh(core_axis_name="c", subcore_axis_name="s")

    spw = nb // NW
    nch = spw // CHUNK

    @functools.partial(
        pl.kernel,
        out_type=jax.ShapeDtypeStruct((nb, FEAT), jnp.float32),
        mesh=mesh,
        scratch_types=[
            pltpu.VMEM((K, spw), jnp.int32),
            pltpu.VMEM((spw, FEAT), jnp.float32),
            pltpu.SemaphoreType.DMA,
            pltpu.SemaphoreType.DMA,
        ],
    )
    def body(idx_hbm, feat_hbm, out_hbm, idx_v, buf, ow, ad):
        wid = lax.axis_index("s") * NC + lax.axis_index("c")
        pltpu.sync_copy(idx_hbm.at[wid], idx_v)

        def seg(j):
            return buf.at[pl.ds(j * CHUNK, CHUNK)]

        def gather(j, k, sem, add):
            return pltpu.async_copy(
                feat_hbm.at[idx_v.at[k, pl.ds(j * CHUNK, CHUNK)]],
                seg(j), sem, add=add)

        for j in range(nch):
            gather(j, 0, ow, False)
        for j in range(nch):
            pltpu.make_async_copy(
                feat_hbm.at[idx_v.at[0, pl.ds(j * CHUNK, CHUNK)]], seg(j), ow).wait()
        for j in range(nch):
            for k in range(1, K):
                gather(j, k, ad, True)
        for j in range(nch):
            for k in range(1, K):
                pltpu.make_async_copy(
                    feat_hbm.at[idx_v.at[k, pl.ds(j * CHUNK, CHUNK)]],
                    seg(j), ad).wait()

        pltpu.sync_copy(buf, out_hbm.at[pl.ds(wid * spw, spw)])

    return body(idx, features)


def _project(combined, Wt):
    def mm(x_ref, w_ref, o_ref):
        o_ref[...] = lax.dot_general(
            x_ref[...], w_ref[...], (((1,), (0,)), ((), ())),
            preferred_element_type=jnp.float32,
        )

    nb = combined.shape[0]
    return pl.pallas_call(
        mm,
        grid=(nb // 1024,),
        in_specs=[
            pl.BlockSpec((1024, FEAT), lambda i: (i, 0)),
            pl.BlockSpec((FEAT, EMB), lambda i: (0, 0)),
        ],
        out_specs=pl.BlockSpec((1024, EMB), lambda i: (i, 0)),
        out_shape=jax.ShapeDtypeStruct((nb, EMB), jnp.float32),
    )(combined, Wt)


NSLICE = 2


def kernel(nodes, neigh_idx, features, W):
    del nodes
    nb = B // NSLICE
    idx = (neigh_idx.astype(jnp.int32)
           .reshape(NSLICE, NW, nb // NW, K)
           .transpose(0, 1, 3, 2))
    Wt = W.T * (1.0 / K)
    outs = []
    for s in range(NSLICE):
        sums = _sc_gather_sum(idx[s], features, nb)
        outs.append(_project(sums, Wt))
    return jnp.concatenate(outs, axis=0)


# speedup vs baseline: 1.0631x; 1.0631x over previous
import functools

import jax
import jax.numpy as jnp
from jax import lax
from jax.experimental import pallas as pl
from jax.experimental.pallas import tpu as pltpu
from jax.experimental.pallas import tpu_sc as plsc

B = 16384
K = 32
FEAT = 128
EMB = 128
NC, NS = 2, 16
NW = NC * NS
SEEDS_PER_W = B // NW
CHUNK = 128
NCH = SEEDS_PER_W // CHUNK


def _sc_gather_sum(idx, features, nb):
    mesh = plsc.VectorSubcoreMesh(core_axis_name="c", subcore_axis_name="s")

    spw = nb // NW
    nch = spw // CHUNK

    @functools.partial(
        pl.kernel,
        out_type=jax.ShapeDtypeStruct((nb, FEAT), jnp.float32),
        mesh=mesh,
        scratch_types=[
            pltpu.VMEM((K, spw), jnp.int32),
            pltpu.VMEM((spw, FEAT), jnp.float32),
            pltpu.SemaphoreType.DMA,
            pltpu.SemaphoreType.DMA,
        ],
    )
    def body(idx_hbm, feat_hbm, out_hbm, idx_v, buf, ow, ad):
        wid = lax.axis_index("s") * NC + lax.axis_index("c")
        pltpu.sync_copy(idx_hbm.at[wid], idx_v)

        def seg(j):
            return buf.at[pl.ds(j * CHUNK, CHUNK)]

        def gather(j, k, sem, add):
            return pltpu.async_copy(
                feat_hbm.at[idx_v.at[k, pl.ds(j * CHUNK, CHUNK)]],
                seg(j), sem, add=add)

        for j in range(nch):
            gather(j, 0, ow, False)
        for j in range(nch):
            pltpu.make_async_copy(
                feat_hbm.at[idx_v.at[0, pl.ds(j * CHUNK, CHUNK)]], seg(j), ow).wait()
        for j in range(nch):
            for k in range(1, K):
                gather(j, k, ad, True)
        for j in range(nch):
            for k in range(1, K):
                pltpu.make_async_copy(
                    feat_hbm.at[idx_v.at[k, pl.ds(j * CHUNK, CHUNK)]],
                    seg(j), ad).wait()

        pltpu.sync_copy(buf, out_hbm.at[pl.ds(wid * spw, spw)])

    return body(idx, features)


def _project(combined, W):
    def mm(x_ref, w_ref, o_ref):
        o_ref[...] = lax.dot_general(
            x_ref[...], w_ref[...], (((1,), (1,)), ((), ())),
            preferred_element_type=jnp.float32,
        ) * (1.0 / K)

    nb = combined.shape[0]
    return pl.pallas_call(
        mm,
        grid=(nb // 1024,),
        in_specs=[
            pl.BlockSpec((1024, FEAT), lambda i: (i, 0)),
            pl.BlockSpec((FEAT, EMB), lambda i: (0, 0)),
        ],
        out_specs=pl.BlockSpec((1024, EMB), lambda i: (i, 0)),
        out_shape=jax.ShapeDtypeStruct((nb, EMB), jnp.float32),
    )(combined, W)


def kernel(nodes, neigh_idx, features, W):
    del nodes
    idx = (neigh_idx.astype(jnp.int32)
           .reshape(NW, SEEDS_PER_W, K)
           .transpose(0, 2, 1))
    sums = _sc_gather_sum(idx, features, B)
    return _project(sums, W)


# speedup vs baseline: 1.1164x; 1.0502x over previous
import functools

import jax
import jax.numpy as jnp
from jax import lax
from jax.experimental import pallas as pl
from jax.experimental.pallas import tpu as pltpu
from jax.experimental.pallas import tpu_sc as plsc

B = 16384
K = 32
FEAT = 128
EMB = 128
NC, NS = 2, 16
NW = NC * NS
SEEDS_PER_W = B // NW
CHUNK = 128
NCH = SEEDS_PER_W // CHUNK


def _sc_gather_sum(idx, features, nb):
    mesh = plsc.VectorSubcoreMesh(core_axis_name="c", subcore_axis_name="s")

    spw = nb // NW
    nch = spw // CHUNK

    @functools.partial(
        pl.kernel,
        out_type=jax.ShapeDtypeStruct((nb, FEAT), jnp.float32),
        mesh=mesh,
        scratch_types=[
            pltpu.VMEM((K, spw), jnp.int32),
            pltpu.VMEM((spw, FEAT), jnp.float32),
            pltpu.SemaphoreType.DMA,
            pltpu.SemaphoreType.DMA,
        ],
    )
    def body(idx_hbm, feat_hbm, out_hbm, idx_v, buf, ow, ad):
        wid = lax.axis_index("s") * NC + lax.axis_index("c")
        pltpu.sync_copy(idx_hbm.at[wid], idx_v)

        def seg(j):
            return buf.at[pl.ds(j * CHUNK, CHUNK)]

        def gather(j, k, sem, add):
            return pltpu.async_copy(
                feat_hbm.at[idx_v.at[k, pl.ds(j * CHUNK, CHUNK)]],
                seg(j), sem, add=add)

        for j in range(nch):
            gather(j, 0, ow, False)
        for j in range(nch):
            pltpu.make_async_copy(
                feat_hbm.at[idx_v.at[0, pl.ds(j * CHUNK, CHUNK)]], seg(j), ow).wait()
        for j in range(nch):
            for k in range(1, K):
                gather(j, k, ad, True)
        for j in range(nch):
            for k in range(1, K):
                pltpu.make_async_copy(
                    feat_hbm.at[idx_v.at[k, pl.ds(j * CHUNK, CHUNK)]],
                    seg(j), ad).wait()

        pltpu.sync_copy(buf, out_hbm.at[pl.ds(wid * spw, spw)])

    return body(idx, features)


def _project(combined, W):
    def mm(x_ref, w_ref, o_ref):
        o_ref[...] = lax.dot_general(
            x_ref[...], w_ref[...], (((1,), (1,)), ((), ())),
            preferred_element_type=jnp.float32,
        ) * (1.0 / K)

    nb = combined.shape[0]
    return pl.pallas_call(
        mm,
        grid=(nb // 4096,),
        in_specs=[
            pl.BlockSpec((4096, FEAT), lambda i: (i, 0)),
            pl.BlockSpec((FEAT, EMB), lambda i: (0, 0)),
        ],
        out_specs=pl.BlockSpec((4096, EMB), lambda i: (i, 0)),
        out_shape=jax.ShapeDtypeStruct((nb, EMB), jnp.float32),
    )(combined, W)


def kernel(nodes, neigh_idx, features, W):
    del nodes
    idx = (neigh_idx.astype(jnp.int32)
           .reshape(NW, SEEDS_PER_W, K)
           .transpose(0, 2, 1))
    sums = _sc_gather_sum(idx, features, B)
    return _project(sums, W)


# speedup vs baseline: 1.1318x; 1.0137x over previous
import functools

import jax
import jax.numpy as jnp
from jax import lax
from jax.experimental import pallas as pl
from jax.experimental.pallas import tpu as pltpu
from jax.experimental.pallas import tpu_sc as plsc

B = 16384
K = 32
FEAT = 128
EMB = 128
NC, NS = 2, 16
NW = NC * NS
SEEDS_PER_W = B // NW
CHUNK = 128
NCH = SEEDS_PER_W // CHUNK


def _sc_gather_sum(idx, features, nb):
    mesh = plsc.VectorSubcoreMesh(core_axis_name="c", subcore_axis_name="s")

    spw = nb // NW
    nch = spw // CHUNK

    @functools.partial(
        pl.kernel,
        out_type=jax.ShapeDtypeStruct((nb, FEAT), jnp.float32),
        mesh=mesh,
        scratch_types=[
            pltpu.VMEM((K, spw), jnp.int32),
            pltpu.VMEM((spw, FEAT), jnp.float32),
            pltpu.SemaphoreType.DMA,
            pltpu.SemaphoreType.DMA,
        ],
    )
    def body(idx_hbm, feat_hbm, out_hbm, idx_v, buf, ow, ad):
        wid = lax.axis_index("s") * NC + lax.axis_index("c")
        pltpu.sync_copy(idx_hbm.at[wid], idx_v)

        def seg(j):
            return buf.at[pl.ds(j * CHUNK, CHUNK)]

        def gather(j, k, sem, add):
            return pltpu.async_copy(
                feat_hbm.at[idx_v.at[k, pl.ds(j * CHUNK, CHUNK)]],
                seg(j), sem, add=add)

        for j in range(nch):
            gather(j, 0, ow, False)
        for j in range(nch):
            pltpu.make_async_copy(
                feat_hbm.at[idx_v.at[0, pl.ds(j * CHUNK, CHUNK)]], seg(j), ow).wait()
        for j in range(nch):
            for k in range(1, K):
                gather(j, k, ad, True)
        for j in range(nch):
            for k in range(1, K):
                pltpu.make_async_copy(
                    feat_hbm.at[idx_v.at[k, pl.ds(j * CHUNK, CHUNK)]],
                    seg(j), ad).wait()

        pltpu.sync_copy(buf, out_hbm.at[pl.ds(wid * spw, spw)])

    return body(idx, features)


def _project(combined, W):
    def mm(x_ref, w_ref, o_ref):
        o_ref[...] = lax.dot_general(
            x_ref[...], w_ref[...], (((1,), (1,)), ((), ())),
            preferred_element_type=jnp.float32,
        ) * (1.0 / K)

    nb = combined.shape[0]
    return pl.pallas_call(
        mm,
        grid=(nb // 8192,),
        in_specs=[
            pl.BlockSpec((8192, FEAT), lambda i: (i, 0)),
            pl.BlockSpec((FEAT, EMB), lambda i: (0, 0)),
        ],
        out_specs=pl.BlockSpec((8192, EMB), lambda i: (i, 0)),
        out_shape=jax.ShapeDtypeStruct((nb, EMB), jnp.float32),
    )(combined, W)


def kernel(nodes, neigh_idx, features, W):
    del nodes
    idx = (neigh_idx.astype(jnp.int32)
           .reshape(NW, SEEDS_PER_W, K)
           .transpose(0, 2, 1))
    sums = _sc_gather_sum(idx, features, B)
    return _project(sums, W)
